# baseline (speedup 1.0000x reference)
import jax
import jax.numpy as jnp
from jax.experimental import pallas as pl
from jax.experimental.pallas import tpu as pltpu

B = 64
NPG = 1536
N = B * NPG
D = 256
C = 512
NB = 1024
TILE = 6144
GRAPHS_PER_TILE = TILE // NPG


def _kernel(sw_ref, bit_probs_ref, h_ref, wp_ref, bp_ref, w1_ref, b1_ref,
            x_ref, mask_ref, gamma_ref, beta_ref, b2_ref, w2_ref,
            out_ref, syn_ref, w1at_ref, w2t_ref):
    i = pl.program_id(0)

    @pl.when(i == 0)
    def _prep():
        w1at_ref[:] = w1_ref[:, :D].T
        w2t_ref[:] = w2_ref[:].T
        p = bit_probs_ref[:]
        llr = jnp.log((p + 1e-08) / (1.0 - p + 1e-08))
        llr = jnp.clip(llr, -10.0, 10.0)
        th = jnp.tanh(0.5 * llr)
        soft_syn = jnp.tanh(0.5 * jax.lax.dot_general(
            th, h_ref[:], (((1,), (1,)), ((), ())),
            preferred_element_type=jnp.float32))
        prob = 0.5 * (1.0 - soft_syn)
        syn_feat = jax.lax.dot_general(
            prob, wp_ref[:], (((1,), (1,)), ((), ())),
            preferred_element_type=jnp.float32) + bp_ref[:]
        syn = jax.lax.dot_general(
            syn_feat, w1_ref[:, D:], (((1,), (1,)), ((), ())),
            preferred_element_type=jnp.float32) + b1_ref[:]
        for s in range(N // TILE):
            blk = syn[s * GRAPHS_PER_TILE:(s + 1) * GRAPHS_PER_TILE, :]
            syn_ref[s * 8:s * 8 + GRAPHS_PER_TILE, :] = blk

    x = x_ref[:]
    h = jnp.dot(x, w1at_ref[:], preferred_element_type=jnp.float32)
    syn_tile = syn_ref[pl.ds(i * 8, 8), :]
    h = h + jnp.concatenate(
        [jnp.broadcast_to(syn_tile[k:k + 1, :], (NPG, D)) for k in range(GRAPHS_PER_TILE)],
        axis=0,
    )
    mu = jnp.mean(h, axis=-1, keepdims=True)
    var = jnp.mean((h - mu) ** 2, axis=-1, keepdims=True)
    h = (h - mu) * jax.lax.rsqrt(var + 1e-05) * gamma_ref[:] + beta_ref[:]
    h = jnp.maximum(h, 0.0)
    enhanced = jnp.dot(h, w2t_ref[:], preferred_element_type=jnp.float32) + b2_ref[:]
    msw = jnp.where(mask_ref[:], sw_ref[0], 0.0)
    out_ref[:] = x + msw * (enhanced - x)


def kernel(node_features, bit_probs, H, var_node_mask, Wp, bp, W1, b1, gamma, beta, W2, b2, syndrome_weight):
    grid = (N // TILE,)
    full = lambda i: (0, 0)
    out = pl.pallas_call(
        _kernel,
        grid=grid,
        in_specs=[
            pl.BlockSpec(memory_space=pltpu.SMEM),
            pl.BlockSpec((B, NB), full),
            pl.BlockSpec((C, NB), full),
            pl.BlockSpec((D // 2, C), full),
            pl.BlockSpec((1, D // 2), full),
            pl.BlockSpec((D, D + D // 2), full),
            pl.BlockSpec((1, D), full),
            pl.BlockSpec((TILE, D), lambda i: (i, 0)),
            pl.BlockSpec((TILE, 1), lambda i: (i, 0)),
            pl.BlockSpec((1, D), full),
            pl.BlockSpec((1, D), full),
            pl.BlockSpec((1, D), full),
            pl.BlockSpec((D, D), full),
        ],
        out_specs=pl.BlockSpec((TILE, D), lambda i: (i, 0)),
        out_shape=jax.ShapeDtypeStruct((N, D), jnp.float32),
        scratch_shapes=[
            pltpu.VMEM((N // TILE * 8, D), jnp.float32),
            pltpu.VMEM((D, D), jnp.float32),
            pltpu.VMEM((D, D), jnp.float32),
        ],
        compiler_params=pltpu.CompilerParams(
            dimension_semantics=("arbitrary",),
        ),
    )(
        syndrome_weight.reshape(1),
        bit_probs,
        H,
        Wp,
        bp.reshape(1, D // 2),
        W1,
        b1.reshape(1, D),
        node_features,
        var_node_mask.reshape(N, 1),
        gamma.reshape(1, D),
        beta.reshape(1, D),
        b2.reshape(1, D),
        W2,
    )
    return out


# speedup vs baseline: 1.0052x; 1.0052x over previous
import jax
import jax.numpy as jnp
from jax.experimental import pallas as pl
from jax.experimental.pallas import tpu as pltpu

B = 64
NPG = 1536
N = B * NPG
D = 256
C = 512
NB = 1024
TILE = 6144
GRAPHS_PER_TILE = TILE // NPG


def _kernel(bit_probs_ref, h_ref, wp_ref, bp_ref, w1_ref, b1_ref,
            x_ref, msw_ref, gamma_ref, beta_ref, b2_ref, w2_ref,
            out_ref, syn_ref, w1at_ref, w2t_ref):
    i = pl.program_id(0)

    @pl.when(i == 0)
    def _prep():
        w1at_ref[:] = w1_ref[:, :D].T
        w2t_ref[:] = w2_ref[:].T
        p = bit_probs_ref[:]
        llr = jnp.log((p + 1e-08) / (1.0 - p + 1e-08))
        llr = jnp.clip(llr, -10.0, 10.0)
        th = jnp.tanh(0.5 * llr)
        soft_syn = jnp.tanh(0.5 * jax.lax.dot_general(
            th, h_ref[:], (((1,), (1,)), ((), ())),
            preferred_element_type=jnp.float32))
        prob = 0.5 * (1.0 - soft_syn)
        syn_feat = jax.lax.dot_general(
            prob, wp_ref[:], (((1,), (1,)), ((), ())),
            preferred_element_type=jnp.float32) + bp_ref[:]
        syn = jax.lax.dot_general(
            syn_feat, w1_ref[:, D:], (((1,), (1,)), ((), ())),
            preferred_element_type=jnp.float32) + b1_ref[:]
        for s in range(N // TILE):
            blk = syn[s * GRAPHS_PER_TILE:(s + 1) * GRAPHS_PER_TILE, :]
            syn_ref[s * 8:s * 8 + GRAPHS_PER_TILE, :] = blk

    x = x_ref[:]
    h = jnp.dot(x, w1at_ref[:], preferred_element_type=jnp.float32)
    syn_tile = syn_ref[pl.ds(i * 8, 8), :]
    h = h + jnp.concatenate(
        [jnp.broadcast_to(syn_tile[k:k + 1, :], (NPG, D)) for k in range(GRAPHS_PER_TILE)],
        axis=0,
    )
    mu = jnp.mean(h, axis=-1, keepdims=True)
    var = jnp.mean((h - mu) ** 2, axis=-1, keepdims=True)
    h = (h - mu) * jax.lax.rsqrt(var + 1e-05) * gamma_ref[:] + beta_ref[:]
    h = jnp.maximum(h, 0.0)
    enhanced = jnp.dot(h, w2t_ref[:], preferred_element_type=jnp.float32) + b2_ref[:]
    out_ref[:] = x + msw_ref[:] * (enhanced - x)


def kernel(node_features, bit_probs, H, var_node_mask, Wp, bp, W1, b1, gamma, beta, W2, b2, syndrome_weight):
    msw = var_node_mask.astype(jnp.float32).reshape(N, 1) * syndrome_weight
    grid = (N // TILE,)
    full = lambda i: (0, 0)
    out = pl.pallas_call(
        _kernel,
        grid=grid,
        in_specs=[
            pl.BlockSpec((B, NB), full),
            pl.BlockSpec((C, NB), full),
            pl.BlockSpec((D // 2, C), full),
            pl.BlockSpec((1, D // 2), full),
            pl.BlockSpec((D, D + D // 2), full),
            pl.BlockSpec((1, D), full),
            pl.BlockSpec((TILE, D), lambda i: (i, 0)),
            pl.BlockSpec((TILE, 1), lambda i: (i, 0)),
            pl.BlockSpec((1, D), full),
            pl.BlockSpec((1, D), full),
            pl.BlockSpec((1, D), full),
            pl.BlockSpec((D, D), full),
        ],
        out_specs=pl.BlockSpec((TILE, D), lambda i: (i, 0)),
        out_shape=jax.ShapeDtypeStruct((N, D), jnp.float32),
        scratch_shapes=[
            pltpu.VMEM((N // TILE * 8, D), jnp.float32),
            pltpu.VMEM((D, D), jnp.float32),
            pltpu.VMEM((D, D), jnp.float32),
        ],
        compiler_params=pltpu.CompilerParams(
            dimension_semantics=("arbitrary",),
        ),
    )(
        bit_probs,
        H,
        Wp,
        bp.reshape(1, D // 2),
        W1,
        b1.reshape(1, D),
        node_features,
        msw,
        gamma.reshape(1, D),
        beta.reshape(1, D),
        b2.reshape(1, D),
        W2,
    )
    return out
